# baseline (speedup 1.0000x reference)
import functools

import jax
import jax.numpy as jnp
from jax import lax
from jax.experimental import pallas as pl
from jax.experimental.pallas import tpu as pltpu
from jax.experimental.pallas import tpu_sc as plsc

N = 10000
E = 320000
D = 128
H = 256
O = 128
OUT = 128

NC = 2
NS = 16
NT = NC * NS
TPT = 320
OUTR = NT * TPT
TPAD = 16
SEG = 2048
ROUNDS = 158
E_PAD = (ROUNDS + 1) * SEG
CAP = SEG + 256
DUMP = CAP - 16
CH_B = 96
MASK14 = (1 << 14) - 1

_DNUMS = lax.GatherDimensionNumbers(
    offset_dims=(), collapsed_slice_dims=(0,), start_index_map=(0,))


def _bcast(v, i):
  idx = jnp.full((16, 1), i, jnp.int32)
  return lax.gather(v, idx, _DNUMS, (1,),
                    mode=lax.GatherScatterMode.PROMISE_IN_BOUNDS)


def _make_segsum(with_cnt: bool):
  out_type = [jax.ShapeDtypeStruct((OUTR, H), jnp.float32)]
  if with_cnt:
    out_type.append(jax.ShapeDtypeStruct((OUTR,), jnp.float32))

  scratch = [
      pltpu.VMEM((TPT, H), jnp.float32),
      pltpu.VMEM((TPT,), jnp.float32),
      pltpu.VMEM((SEG,), jnp.int32),
      pltpu.VMEM((SEG,), jnp.int32),
      pltpu.VMEM((CAP,), jnp.int32),
      pltpu.VMEM((CAP,), jnp.int32),
      pltpu.VMEM((CH_B,), jnp.int32),
      pltpu.VMEM((CH_B, H), jnp.float32),
      pltpu.SemaphoreType.DMA,
      pltpu.SemaphoreType.DMA,
  ]

  mesh = plsc.VectorSubcoreMesh(core_axis_name="c", subcore_axis_name="s")

  @functools.partial(
      pl.kernel, out_type=tuple(out_type), mesh=mesh, scratch_types=scratch,
      compiler_params=pltpu.CompilerParams(needs_layout_passes=False))
  def segsum(t_hbm, ep_hbm, *refs):
    if with_cnt:
      agg_out, cnt_out = refs[0], refs[1]
      refs = refs[2:]
    else:
      agg_out = refs[0]
      cnt_out = None
      refs = refs[1:]
    acc, cnt, sega, segb, lsta, lstb, stage, rows, ssem, gsem = refs

    c = lax.axis_index("c")
    s = lax.axis_index("s")
    w = s * NC + c
    lo = w * TPT
    lane = lax.iota(jnp.int32, 16)
    zero16 = jnp.zeros((16,), jnp.float32)
    zero16i = jnp.zeros((16,), jnp.int32)

    def zacc(j, _):
      for k in range(H // 16):
        acc[j, pl.ds(k * 16, 16)] = zero16
      return 0
    lax.fori_loop(0, TPT, zacc, 0)
    for j in range(TPT // 16):
      cnt[pl.ds(j * 16, 16)] = zero16

    def phase_a(segbuf, lstbuf):
      def pa(j, kvec):
        p = segbuf[pl.ds(j * 16, 16)]
        sv = p & MASK14
        d = lax.shift_right_logical(p, 14)
        m = (d >= lo) & (d < lo + TPT)
        mi = m.astype(jnp.int32)
        csum = plsc.cumsum(mi)
        pos = kvec + csum - 1
        idx = jnp.where(m, pos, DUMP + lane)
        ld = d - lo
        pk = sv | lax.shift_left(jnp.where(m, ld, 0), 14)
        plsc.store_scatter(lstbuf, [idx], pk)
        if with_cnt:
          plsc.addupdate_scatter(cnt, [jnp.where(m, ld, 0)],
                                 m.astype(jnp.float32))
        return kvec + _bcast(csum, 15)

      kvec = lax.fori_loop(0, SEG // 16, pa, zero16i)
      return jnp.sum(jnp.where(lane == 15, kvec, 0))

    def fire_chunk(lstbuf, base, K):
      for j in range(CH_B // 16):
        posv = base + j * 16 + lane
        p = lstbuf[pl.ds(base + j * 16, 16)]
        stage[pl.ds(j * 16, 16)] = jnp.where(posv < K, p & MASK14, N)
      return pltpu.async_copy(t_hbm.at[stage], rows, gsem)

    def wait_gather():
      pltpu.make_async_copy(t_hbm.at[stage], rows, gsem).wait()

    def acc_chunk(lstbuf, base, K):
      def aj(j, _):
        posv = base + j * 16 + lane
        p = lstbuf[pl.ds(base + j * 16, 16)]
        ldv = jnp.where(posv < K, lax.shift_right_logical(p, 14), 0)
        for e16 in range(16):
          e = j * 16 + e16
          ldb = _bcast(ldv, e16)
          for k in range(H // 16):
            x = rows[e, pl.ds(k * 16, 16)]
            plsc.addupdate_scatter(acc, [ldb, lane + k * 16], x)
        return 0
      lax.fori_loop(0, CH_B // 16, aj, 0)

    def extras(lstbuf, K):
      nb = lax.shift_right_logical((K + (CH_B - 1)) * 683, 16)

      def ex(i, _):
        fire_chunk(lstbuf, i * CH_B, K).wait()
        acc_chunk(lstbuf, i * CH_B, K)
        return 0
      lax.fori_loop(1, nb, ex, 0)

    pltpu.sync_copy(ep_hbm.at[pl.ds(0, SEG)], sega)

    def body2(r2, K_prev):
      re = 2 * r2
      ro = re + 1

      @pl.when(re > 0)
      def _():
        pltpu.make_async_copy(ep_hbm.at[pl.ds(re * SEG, SEG)], sega,
                              ssem).wait()
      K_e = phase_a(sega, lsta)
      pltpu.async_copy(ep_hbm.at[pl.ds(ro * SEG, SEG)], segb, ssem)

      @pl.when(re > 0)
      def _():
        wait_gather()
        acc_chunk(lstb, 0, K_prev)
        extras(lstb, K_prev)
      fire_chunk(lsta, 0, K_e)

      pltpu.make_async_copy(ep_hbm.at[pl.ds(ro * SEG, SEG)], segb,
                            ssem).wait()
      K_o = phase_a(segb, lstb)
      pltpu.async_copy(ep_hbm.at[pl.ds((ro + 1) * SEG, SEG)], sega, ssem)

      wait_gather()
      acc_chunk(lsta, 0, K_e)
      extras(lsta, K_e)
      fire_chunk(lstb, 0, K_o)
      return K_o

    K_last = lax.fori_loop(0, ROUNDS // 2, body2, jnp.int32(0))

    wait_gather()
    acc_chunk(lstb, 0, K_last)
    extras(lstb, K_last)
    pltpu.make_async_copy(ep_hbm.at[pl.ds(ROUNDS * SEG, SEG)], sega,
                          ssem).wait()

    pltpu.sync_copy(acc, agg_out.at[pl.ds(lo, TPT)])
    if with_cnt:
      pltpu.sync_copy(cnt, cnt_out.at[pl.ds(lo, TPT)])

  return segsum


_segsum_cnt = _make_segsum(True)
_segsum = _make_segsum(False)


def _tc1_body(x_ref, g_ref, b_ref, q_ref, qb_ref, h_ref, t_ref):
  x = x_ref[...]
  mean = jnp.mean(x, axis=0, keepdims=True)
  var = jnp.mean((x - mean) ** 2, axis=0, keepdims=True)
  h = (x - mean) / jnp.sqrt(var + 1e-5) * g_ref[...] + b_ref[...]
  h_ref[...] = h
  t_ref[0:N, :] = jnp.maximum(h @ q_ref[...] + qb_ref[...], 0.0)
  t_ref[N:N + TPAD, :] = jnp.zeros((TPAD, H), jnp.float32)


def _tc2_body(h_ref, agg_ref, cnt_ref, wa_ref, wb_ref, bias_ref,
              q_ref, qb_ref, h1_ref, t1_ref):
  h = h_ref[...]
  cnt = jnp.maximum(cnt_ref[...], 1.0)
  hn = agg_ref[0:N, :] / cnt
  z = jnp.maximum(h @ wa_ref[...] + hn @ wb_ref[...] + bias_ref[...], 0.0)
  nrm = jnp.sqrt(jnp.sum(z * z, axis=1, keepdims=True))
  h1 = z / jnp.maximum(nrm, 1e-12)
  h1_ref[...] = h1
  t1_ref[0:N, :] = jnp.maximum(h1 @ q_ref[...] + qb_ref[...], 0.0)
  t1_ref[N:N + TPAD, :] = jnp.zeros((TPAD, H), jnp.float32)


def _tc3_body(h_ref, agg_ref, cnt_ref, wa_ref, wb_ref, bias_ref,
              gw_ref, gb_ref, gs_ref, og_ref, ob_ref, out_ref):
  h = h_ref[...]
  cnt = jnp.maximum(cnt_ref[...], 1.0)
  hn = agg_ref[0:N, :] / cnt
  z = jnp.maximum(h @ wa_ref[...] + hn @ wb_ref[...] + bias_ref[...], 0.0)
  nrm = jnp.sqrt(jnp.sum(z * z, axis=1, keepdims=True))
  h2 = z / jnp.maximum(nrm, 1e-12)
  y = gs_ref[0, 0] * jnp.maximum(h2 @ gw_ref[...] + gb_ref[...], 0.0)
  mean = jnp.mean(y, axis=0, keepdims=True)
  var = jnp.mean((y - mean) ** 2, axis=0, keepdims=True)
  out_ref[...] = (y - mean) / jnp.sqrt(var + 1e-5) * og_ref[...] + ob_ref[...]


def _tc1(x, gamma, beta, Q, qb):
  return pl.pallas_call(
      _tc1_body,
      out_shape=(jax.ShapeDtypeStruct((N, D), jnp.float32),
                 jax.ShapeDtypeStruct((N + TPAD, H), jnp.float32)),
  )(x, gamma, beta, Q, qb)


def _tc2(h, agg, cnt_col, Wa, Wb, wb, Q, qb):
  return pl.pallas_call(
      _tc2_body,
      out_shape=(jax.ShapeDtypeStruct((N, O), jnp.float32),
                 jax.ShapeDtypeStruct((N + TPAD, H), jnp.float32)),
  )(h, agg, cnt_col, Wa, Wb, wb, Q, qb)


def _tc3(h, agg, cnt_col, Wa, Wb, wb, Gw, Gb, gs, og, ob):
  return pl.pallas_call(
      _tc3_body,
      out_shape=jax.ShapeDtypeStruct((N, OUT), jnp.float32),
  )(h, agg, cnt_col, Wa, Wb, wb, Gw, Gb, gs, og, ob)


def kernel(inputs, edge_index, bn_in_gamma, bn_in_beta, Q0, qb0, W0, wb0,
           Q1, qb1, W1, wb1, Gw, Gb, g_scalar, bn_out_gamma, bn_out_beta):
  src = edge_index[0]
  dst = edge_index[1]
  pad = E_PAD - E
  padv = jnp.full((pad,), N | (N << 14), jnp.int32)
  ep = jnp.concatenate([src | (dst << 14), padv])

  row = lambda v: v.reshape(1, -1)

  h0, t0 = _tc1(inputs, row(bn_in_gamma), row(bn_in_beta), Q0, row(qb0))
  agg0, cnt_v = _segsum_cnt(t0, ep)
  cnt_col = cnt_v[:N].reshape(N, 1)
  h1, t1 = _tc2(h0, agg0, cnt_col, W0[:D], W0[D:], row(wb0), Q1, row(qb1))
  (agg1,) = _segsum(t1, ep)
  out = _tc3(h1, agg1, cnt_col, W1[:O], W1[O:], row(wb1), Gw, row(Gb),
             g_scalar.reshape(1, 1), row(bn_out_gamma), row(bn_out_beta))
  return out


# speedup vs baseline: 1.0940x; 1.0940x over previous
import functools

import jax
import jax.numpy as jnp
from jax import lax
from jax.experimental import pallas as pl
from jax.experimental.pallas import tpu as pltpu
from jax.experimental.pallas import tpu_sc as plsc

N = 10000
E = 320000
D = 128
H = 256
O = 128
OUT = 128

NC = 2
NS = 16
NT = NC * NS
TPT = 320
OUTR = NT * TPT
TPAD = 16
SEG = 2048
ROUNDS = (E + SEG - 1) // SEG
E_PAD = ROUNDS * SEG
CAP = SEG + 256
DUMP = CAP - 16
CH_B = 64

_DNUMS = lax.GatherDimensionNumbers(
    offset_dims=(), collapsed_slice_dims=(0,), start_index_map=(0,))


def _bcast(v, i):
  idx = jnp.full((16, 1), i, jnp.int32)
  return lax.gather(v, idx, _DNUMS, (1,),
                    mode=lax.GatherScatterMode.PROMISE_IN_BOUNDS)


def _make_segsum(with_cnt: bool):
  out_type = [jax.ShapeDtypeStruct((OUTR, H), jnp.float32)]
  if with_cnt:
    out_type.append(jax.ShapeDtypeStruct((OUTR,), jnp.float32))

  scratch = [
      pltpu.VMEM((TPT, H), jnp.float32),
      pltpu.VMEM((TPT,), jnp.float32),
      pltpu.VMEM((SEG,), jnp.int32),
      pltpu.VMEM((SEG,), jnp.int32),
      pltpu.VMEM((CAP,), jnp.int32),
      pltpu.VMEM((CAP,), jnp.int32),
      pltpu.VMEM((CH_B,), jnp.int32),
      pltpu.VMEM((CH_B, H), jnp.float32),
      pltpu.SemaphoreType.DMA,
  ]

  mesh = plsc.VectorSubcoreMesh(core_axis_name="c", subcore_axis_name="s")

  @functools.partial(
      pl.kernel, out_type=tuple(out_type), mesh=mesh, scratch_types=scratch,
      compiler_params=pltpu.CompilerParams(needs_layout_passes=False))
  def segsum(t_hbm, src_hbm, dst_hbm, *refs):
    if with_cnt:
      agg_out, cnt_out = refs[0], refs[1]
      refs = refs[2:]
    else:
      agg_out = refs[0]
      cnt_out = None
      refs = refs[1:]
    acc, cnt, segs, segd, csrc, cld, stage, rows, sem = refs

    c = lax.axis_index("c")
    s = lax.axis_index("s")
    w = s * NC + c
    lo = w * TPT
    lane = lax.iota(jnp.int32, 16)
    zero16 = jnp.zeros((16,), jnp.float32)
    zero16i = jnp.zeros((16,), jnp.int32)

    def zacc(j, _):
      for k in range(H // 16):
        acc[j, pl.ds(k * 16, 16)] = zero16
      return 0
    lax.fori_loop(0, TPT, zacc, 0)
    for j in range(TPT // 16):
      cnt[pl.ds(j * 16, 16)] = zero16

    def round_body(r, _):
      pltpu.sync_copy(src_hbm.at[pl.ds(r * SEG, SEG)], segs)
      pltpu.sync_copy(dst_hbm.at[pl.ds(r * SEG, SEG)], segd)

      def pa(j, kvec):
        d = segd[pl.ds(j * 16, 16)]
        sv = segs[pl.ds(j * 16, 16)]
        m = (d >= lo) & (d < lo + TPT)
        mi = m.astype(jnp.int32)
        csum = plsc.cumsum(mi)
        pos = kvec + csum - 1
        idx = jnp.where(m, pos, DUMP + lane)
        ld = jnp.where(m, d - lo, 0)
        plsc.store_scatter(csrc, [idx], sv)
        plsc.store_scatter(cld, [idx], ld)
        if with_cnt:
          plsc.addupdate_scatter(cnt, [ld], m.astype(jnp.float32))
        return kvec + _bcast(csum, 15)

      kvec = lax.fori_loop(0, SEG // 16, pa, zero16i)
      K = jnp.sum(jnp.where(lane == 15, kvec, 0))
      nb = lax.shift_right_logical(K + (CH_B - 1), 6)

      def pb(i, _):
        base = i * CH_B
        ldvs = []
        for j in range(CH_B // 16):
          posv = base + j * 16 + lane
          inb = posv < K
          sv = csrc[pl.ds(base + j * 16, 16)]
          ldv = cld[pl.ds(base + j * 16, 16)]
          stage[pl.ds(j * 16, 16)] = jnp.where(inb, sv, N)
          ldvs.append(jnp.where(inb, ldv, 0))
        pltpu.async_copy(t_hbm.at[stage], rows, sem).wait()
        for j in range(CH_B // 16):
          ldv = ldvs[j]
          for e16 in range(16):
            e = j * 16 + e16
            ldb = _bcast(ldv, e16)
            for k in range(H // 16):
              x = rows[e, pl.ds(k * 16, 16)]
              plsc.addupdate_scatter(acc, [ldb, lane + k * 16], x)
        return 0

      lax.fori_loop(0, nb, pb, 0)
      return 0

    lax.fori_loop(0, ROUNDS, round_body, 0)

    pltpu.sync_copy(acc, agg_out.at[pl.ds(lo, TPT)])
    if with_cnt:
      pltpu.sync_copy(cnt, cnt_out.at[pl.ds(lo, TPT)])

  return segsum


_segsum_cnt = _make_segsum(True)
_segsum = _make_segsum(False)


def _tc1_body(x_ref, g_ref, b_ref, q_ref, qb_ref, h_ref, t_ref):
  x = x_ref[...]
  mean = jnp.mean(x, axis=0, keepdims=True)
  var = jnp.mean((x - mean) ** 2, axis=0, keepdims=True)
  h = (x - mean) / jnp.sqrt(var + 1e-5) * g_ref[...] + b_ref[...]
  h_ref[...] = h
  t_ref[0:N, :] = jnp.maximum(h @ q_ref[...] + qb_ref[...], 0.0)
  t_ref[N:N + TPAD, :] = jnp.zeros((TPAD, H), jnp.float32)


def _tc2_body(h_ref, agg_ref, cnt_ref, wa_ref, wb_ref, bias_ref,
              q_ref, qb_ref, h1_ref, t1_ref):
  h = h_ref[...]
  cnt = jnp.maximum(cnt_ref[...], 1.0)
  hn = agg_ref[0:N, :] / cnt
  z = jnp.maximum(h @ wa_ref[...] + hn @ wb_ref[...] + bias_ref[...], 0.0)
  nrm = jnp.sqrt(jnp.sum(z * z, axis=1, keepdims=True))
  h1 = z / jnp.maximum(nrm, 1e-12)
  h1_ref[...] = h1
  t1_ref[0:N, :] = jnp.maximum(h1 @ q_ref[...] + qb_ref[...], 0.0)
  t1_ref[N:N + TPAD, :] = jnp.zeros((TPAD, H), jnp.float32)


def _tc3_body(h_ref, agg_ref, cnt_ref, wa_ref, wb_ref, bias_ref,
              gw_ref, gb_ref, gs_ref, og_ref, ob_ref, out_ref):
  h = h_ref[...]
  cnt = jnp.maximum(cnt_ref[...], 1.0)
  hn = agg_ref[0:N, :] / cnt
  z = jnp.maximum(h @ wa_ref[...] + hn @ wb_ref[...] + bias_ref[...], 0.0)
  nrm = jnp.sqrt(jnp.sum(z * z, axis=1, keepdims=True))
  h2 = z / jnp.maximum(nrm, 1e-12)
  y = gs_ref[0, 0] * jnp.maximum(h2 @ gw_ref[...] + gb_ref[...], 0.0)
  mean = jnp.mean(y, axis=0, keepdims=True)
  var = jnp.mean((y - mean) ** 2, axis=0, keepdims=True)
  out_ref[...] = (y - mean) / jnp.sqrt(var + 1e-5) * og_ref[...] + ob_ref[...]


def _tc1(x, gamma, beta, Q, qb):
  return pl.pallas_call(
      _tc1_body,
      out_shape=(jax.ShapeDtypeStruct((N, D), jnp.float32),
                 jax.ShapeDtypeStruct((N + TPAD, H), jnp.float32)),
  )(x, gamma, beta, Q, qb)


def _tc2(h, agg, cnt_col, Wa, Wb, wb, Q, qb):
  return pl.pallas_call(
      _tc2_body,
      out_shape=(jax.ShapeDtypeStruct((N, O), jnp.float32),
                 jax.ShapeDtypeStruct((N + TPAD, H), jnp.float32)),
  )(h, agg, cnt_col, Wa, Wb, wb, Q, qb)


def _tc3(h, agg, cnt_col, Wa, Wb, wb, Gw, Gb, gs, og, ob):
  return pl.pallas_call(
      _tc3_body,
      out_shape=jax.ShapeDtypeStruct((N, OUT), jnp.float32),
  )(h, agg, cnt_col, Wa, Wb, wb, Gw, Gb, gs, og, ob)


def kernel(inputs, edge_index, bn_in_gamma, bn_in_beta, Q0, qb0, W0, wb0,
           Q1, qb1, W1, wb1, Gw, Gb, g_scalar, bn_out_gamma, bn_out_beta):
  src = edge_index[0]
  dst = edge_index[1]
  pad = E_PAD - E
  src_p = jnp.concatenate([src, jnp.full((pad,), N, jnp.int32)])
  dst_p = jnp.concatenate([dst, jnp.full((pad,), N, jnp.int32)])

  row = lambda v: v.reshape(1, -1)

  h0, t0 = _tc1(inputs, row(bn_in_gamma), row(bn_in_beta), Q0, row(qb0))
  agg0, cnt_v = _segsum_cnt(t0, src_p, dst_p)
  cnt_col = cnt_v[:N].reshape(N, 1)
  h1, t1 = _tc2(h0, agg0, cnt_col, W0[:D], W0[D:], row(wb0), Q1, row(qb1))
  (agg1,) = _segsum(t1, src_p, dst_p)
  out = _tc3(h1, agg1, cnt_col, W1[:O], W1[O:], row(wb1), Gw, row(Gb),
             g_scalar.reshape(1, 1), row(bn_out_gamma), row(bn_out_beta))
  return out
